# Initial kernel scaffold; baseline (speedup 1.0000x reference)
import jax
import jax.numpy as jnp
from jax.experimental import pallas as pl


def kernel(x, edge_index, W1, W2):
    raise NotImplementedError("write your pallas kernel here")


# speedup vs baseline: 3.2692x; 3.2692x over previous
import functools

import jax
import jax.numpy as jnp
from jax import lax
from jax.experimental import pallas as pl
from jax.experimental.pallas import tpu as pltpu
from jax.experimental.pallas import tpu_sc as plsc

N = 10000
D_IN = 128
D_HID = 128
D_OUT = 64
E = 320000

NC = 2
NS = 16
L = 16

N_PAD = 10240
DUMMY = N
E_CH = 128
G_CH = 8
T_CH = -(-E // (NS * E_CH * G_CH)) * G_CH
E_PAD = T_CH * NS * E_CH
T_DEG = T_CH // 2

_MESH = dict(core_axis_name="c", subcore_axis_name="s", num_cores=NC,
             num_subcores=NS)


def _deg_body(dst_hbm, ones_hbm, zeros_hbm, out_hbm,
              dst_v, ones_v, zbuf_v, deg_sh):
    c = lax.axis_index("c")
    s = lax.axis_index("s")
    w = s * NC + c

    pltpu.sync_copy(ones_hbm, ones_v)
    pltpu.sync_copy(zeros_hbm, zbuf_v)
    pltpu.sync_copy(zbuf_v, deg_sh.at[pl.ds(s * (N_PAD // NS), N_PAD // NS)])

    def chunk(g, carry):
        pltpu.sync_copy(dst_hbm.at[w, g], dst_v)
        pltpu.sync_copy(ones_v, deg_sh.at[dst_v], add=True)
        return carry
    lax.fori_loop(0, T_DEG, chunk, 0)

    pltpu.sync_copy(deg_sh.at[pl.ds(s * (N_PAD // NS), N_PAD // NS)],
                    out_hbm.at[c, pl.ds(s * (N_PAD // NS), N_PAD // NS)])


_deg_kernel = functools.partial(
    pl.kernel,
    out_type=jax.ShapeDtypeStruct((NC, N_PAD, L), jnp.float32),
    mesh=plsc.VectorSubcoreMesh(**_MESH),
    scratch_types=[
        pltpu.VMEM((E_CH,), jnp.int32),
        pltpu.VMEM((E_CH, L), jnp.float32),
        pltpu.VMEM((N_PAD // NS, L), jnp.float32),
        pltpu.VMEM_SHARED((N_PAD, L), jnp.float32),
    ],
)(_deg_body)


_RCH = 640


def _min_body(x_hbm, out_hbm, buf_v):
    c = lax.axis_index("c")
    s = lax.axis_index("s")

    @pl.when(s < NS - 1)
    def _():
        r0 = s * _RCH
        pltpu.sync_copy(x_hbm.at[pl.ds(r0, _RCH)], buf_v)
        pltpu.sync_copy(buf_v, out_hbm.at[c, pl.ds(r0, _RCH)])

    @pl.when(s == NS - 1)
    def _():
        r0 = (NS - 1) * _RCH
        nt = N - r0
        pltpu.sync_copy(x_hbm.at[pl.ds(r0, nt)], buf_v.at[pl.ds(0, nt)])
        pltpu.sync_copy(buf_v.at[pl.ds(0, nt)], out_hbm.at[c, pl.ds(r0, nt)])


_min_kernel = functools.partial(
    pl.kernel,
    out_type=jax.ShapeDtypeStruct((NC, N, D_IN), jnp.float32),
    mesh=plsc.VectorSubcoreMesh(**_MESH),
    scratch_types=[
        pltpu.VMEM((640, D_IN), jnp.float32),
    ],
)(_min_body)


def _gat_body(ch, y_hbm, src_hbm, out_hbm, idx_v, rows_v, sem):
    c = lax.axis_index("c")
    s = lax.axis_index("s")
    w = s * NC + c

    def chunk(t, carry):
        pltpu.sync_copy(src_hbm.at[w, t], idx_v)
        pltpu.async_copy(y_hbm.at[idx_v], rows_v, sem).wait()
        pltpu.sync_copy(rows_v, out_hbm.at[w, t])
        return carry
    lax.fori_loop(0, T_DEG, chunk, 0)


def _make_gat(width):
    return functools.partial(
        pl.kernel,
        out_type=jax.ShapeDtypeStruct((NC * NS, T_DEG, E_CH, width),
                                      jnp.float32),
        mesh=plsc.VectorSubcoreMesh(**_MESH),
        scratch_types=[
            pltpu.VMEM((E_CH,), jnp.int32),
            pltpu.VMEM((E_CH, width), jnp.float32),
            pltpu.SemaphoreType.DMA,
        ],
    )(functools.partial(_gat_body, width))


_gat1 = _make_gat(D_HID)
_gat2 = _make_gat(D_OUT)


def _prop_body(ch, y_hbm, src_hbm, dst_hbm, out_hbm,
               src_v, dst_v, rows_v, y_sh, acc_sh, sem):
    c = lax.axis_index("c")
    s = lax.axis_index("s")
    @pl.when(s < NS - 1)
    def _():
        r0 = s * _RCH
        pltpu.sync_copy(y_hbm.at[c, pl.ds(r0, _RCH)], y_sh.at[pl.ds(r0, _RCH)])
        pltpu.sync_copy(y_hbm.at[c, pl.ds(r0, _RCH)],
                        acc_sh.at[pl.ds(r0, _RCH)])

    @pl.when(s == NS - 1)
    def _():
        r0 = (NS - 1) * _RCH
        nt = N - r0
        pltpu.sync_copy(y_hbm.at[c, pl.ds(r0, nt)], y_sh.at[pl.ds(r0, nt)])
        pltpu.sync_copy(y_hbm.at[c, pl.ds(r0, nt)], acc_sh.at[pl.ds(r0, nt)])

    plsc.subcore_barrier()

    def chunk(t, carry):
        pltpu.sync_copy(src_hbm.at[s, t], src_v)
        pltpu.sync_copy(dst_hbm.at[s, t], dst_v)
        pltpu.async_copy(y_sh.at[src_v], rows_v, sem).wait()
        pltpu.sync_copy(rows_v, acc_sh.at[dst_v], add=True)
        return carry
    lax.fori_loop(0, T_CH, chunk, 0)

    plsc.subcore_barrier()

    @pl.when(s < NS - 1)
    def _():
        r0 = s * _RCH
        pltpu.sync_copy(acc_sh.at[pl.ds(r0, _RCH)],
                        out_hbm.at[c, pl.ds(r0, _RCH)])

    @pl.when(s == NS - 1)
    def _():
        r0 = (NS - 1) * _RCH
        nt = N - r0
        pltpu.sync_copy(acc_sh.at[pl.ds(r0, nt)], out_hbm.at[c, pl.ds(r0, nt)])


def _make_prop(width):
    ch = width // NC
    return functools.partial(
        pl.kernel,
        out_type=jax.ShapeDtypeStruct((NC, N, ch), jnp.float32),
        mesh=plsc.VectorSubcoreMesh(**_MESH),
        scratch_types=[
            pltpu.VMEM((E_CH,), jnp.int32),
            pltpu.VMEM((E_CH,), jnp.int32),
            pltpu.VMEM((E_CH, ch), jnp.float32),
            pltpu.VMEM_SHARED((N_PAD, ch), jnp.float32),
            pltpu.VMEM_SHARED((N_PAD, ch), jnp.float32),
            pltpu.SemaphoreType.DMA,
        ],
    )(functools.partial(_prop_body, ch))


_prop1 = _make_prop(D_HID)
_prop2 = _make_prop(D_OUT)

_MB = 2000


def _m1s_body(x_ref, w_ref, h_ref, y_ref, d_ref):
    deg = jnp.sum(h_ref[...], axis=1, keepdims=True) + 1.0
    dinv = lax.rsqrt(deg)
    y_ref[...] = jnp.dot(x_ref[...], w_ref[...],
                         preferred_element_type=jnp.float32) * dinv
    d_ref[...] = dinv


_m1s = pl.pallas_call(
    _m1s_body,
    grid=(N // _MB,),
    in_specs=[
        pl.BlockSpec((_MB, D_IN), lambda i: (i, 0)),
        pl.BlockSpec((D_IN, D_HID), lambda i: (0, 0)),
        pl.BlockSpec((_MB, NC), lambda i: (i, 0)),
    ],
    out_specs=[
        pl.BlockSpec((_MB, D_HID), lambda i: (i, 0)),
        pl.BlockSpec((_MB, 1), lambda i: (i, 0)),
    ],
    out_shape=[
        jax.ShapeDtypeStruct((N, D_HID), jnp.float32),
        jax.ShapeDtypeStruct((N, 1), jnp.float32),
    ],
)


def _m2s_body(a_ref, d_ref, w_ref, y_ref):
    dinv = d_ref[...]
    h = jnp.maximum(a_ref[...] * dinv, 0.0)
    y_ref[...] = jnp.dot(h, w_ref[...],
                         preferred_element_type=jnp.float32) * dinv


_m2s = pl.pallas_call(
    _m2s_body,
    grid=(N // _MB,),
    in_specs=[
        pl.BlockSpec((_MB, D_HID), lambda i: (i, 0)),
        pl.BlockSpec((_MB, 1), lambda i: (i, 0)),
        pl.BlockSpec((D_HID, D_OUT), lambda i: (0, 0)),
    ],
    out_specs=pl.BlockSpec((_MB, D_OUT), lambda i: (i, 0)),
    out_shape=jax.ShapeDtypeStruct((N, D_OUT), jnp.float32),
)


def _m3s_body(a_ref, d_ref, o_ref):
    o = a_ref[...] * d_ref[...]
    m = jnp.max(o, axis=1, keepdims=True)
    o_ref[...] = o - m - jnp.log(jnp.sum(jnp.exp(o - m), axis=1,
                                         keepdims=True))


_m3s = pl.pallas_call(
    _m3s_body,
    grid=(N // _MB,),
    in_specs=[
        pl.BlockSpec((_MB, D_OUT), lambda i: (i, 0)),
        pl.BlockSpec((_MB, 1), lambda i: (i, 0)),
    ],
    out_specs=pl.BlockSpec((_MB, D_OUT), lambda i: (i, 0)),
    out_shape=jax.ShapeDtypeStruct((N, D_OUT), jnp.float32),
)


def kernel(x, edge_index, W1, W2):
    src = edge_index[0].astype(jnp.int32)
    dst = edge_index[1].astype(jnp.int32)
    pad = jnp.full((E_PAD - E,), DUMMY, jnp.int32)
    srcp = jnp.concatenate([src, pad]).reshape(NS, T_CH, E_CH)
    dstp = jnp.concatenate([dst, pad]).reshape(NS, T_CH, E_CH)
    dst_deg = jnp.concatenate([dst, pad]).reshape(NC * NS, T_DEG, E_CH)

    src_g = jnp.concatenate([src, pad]).reshape(NC * NS, T_DEG, E_CH)
    dst_flat = jnp.concatenate([dst, pad])

    degj = jnp.zeros((N,), jnp.float32).at[dst].add(1.0)
    hist2 = jnp.stack([degj, jnp.zeros_like(degj)], axis=1)

    def _scatter(y, msgs):
        return y + jnp.zeros_like(y).at[dst_flat].add(
            msgs.reshape(E_PAD, -1)[:, :y.shape[1]], mode="drop")

    zpad1 = jnp.zeros((N_PAD - N, D_HID), jnp.float32)

    y1, dinv = _m1s(x, W1, hist2)
    msgs1 = _gat1(jnp.concatenate([y1, zpad1]), src_g)
    acc1f = _scatter(y1, msgs1)

    y2 = _m2s(acc1f, dinv, W2)
    y2w = jnp.concatenate(
        [y2, jnp.zeros((N, D_HID - D_OUT), jnp.float32)], axis=1)
    msgs2 = _gat1(jnp.concatenate([y2w, zpad1]), src_g)
    acc2f = _scatter(y2, msgs2)

    return _m3s(acc2f, dinv)


# speedup vs baseline: 3.6755x; 1.1243x over previous
import functools

import jax
import jax.numpy as jnp
from jax import lax
from jax.experimental import pallas as pl
from jax.experimental.pallas import tpu as pltpu
from jax.experimental.pallas import tpu_sc as plsc

N = 10000
D_IN = 128
D_HID = 128
D_OUT = 64
E = 320000

NC = 2
NS = 16
L = 16

N_PAD = 10240
DUMMY = N
E_CH = 512
T_EDG = 20
E_PAD = NC * NS * T_EDG * E_CH

_MESH = dict(core_axis_name="c", subcore_axis_name="s", num_cores=NC,
             num_subcores=NS)


def _gat_body(y_hbm, src_hbm, out_hbm, idx_v, rows_v, sem):
    c = lax.axis_index("c")
    s = lax.axis_index("s")
    w = s * NC + c

    def chunk(t, carry):
        pltpu.sync_copy(src_hbm.at[w, t], idx_v)
        pltpu.async_copy(y_hbm.at[idx_v], rows_v, sem).wait()
        pltpu.sync_copy(rows_v, out_hbm.at[w, t])
        return carry
    lax.fori_loop(0, T_EDG, chunk, 0)


_gat = functools.partial(
    pl.kernel,
    out_type=jax.ShapeDtypeStruct((NC * NS, T_EDG, E_CH, D_HID),
                                  jnp.float32),
    mesh=plsc.VectorSubcoreMesh(**_MESH),
    scratch_types=[
        pltpu.VMEM((E_CH,), jnp.int32),
        pltpu.VMEM((E_CH, D_HID), jnp.float32),
        pltpu.SemaphoreType.DMA,
    ],
)(_gat_body)


_MB = 2000


def _m1s_body(x_ref, w_ref, h_ref, y_ref, d_ref):
    deg = jnp.sum(h_ref[...], axis=1, keepdims=True) + 1.0
    dinv = lax.rsqrt(deg)
    y_ref[...] = jnp.dot(x_ref[...], w_ref[...],
                         preferred_element_type=jnp.float32) * dinv
    d_ref[...] = dinv


_m1s = pl.pallas_call(
    _m1s_body,
    grid=(N // _MB,),
    in_specs=[
        pl.BlockSpec((_MB, D_IN), lambda i: (i, 0)),
        pl.BlockSpec((D_IN, D_HID), lambda i: (0, 0)),
        pl.BlockSpec((_MB, NC), lambda i: (i, 0)),
    ],
    out_specs=[
        pl.BlockSpec((_MB, D_HID), lambda i: (i, 0)),
        pl.BlockSpec((_MB, 1), lambda i: (i, 0)),
    ],
    out_shape=[
        jax.ShapeDtypeStruct((N, D_HID), jnp.float32),
        jax.ShapeDtypeStruct((N, 1), jnp.float32),
    ],
)


def _m2s_body(a_ref, d_ref, w_ref, y_ref):
    dinv = d_ref[...]
    h = jnp.maximum(a_ref[...] * dinv, 0.0)
    y_ref[...] = jnp.dot(h, w_ref[...],
                         preferred_element_type=jnp.float32) * dinv


_m2s = pl.pallas_call(
    _m2s_body,
    grid=(N // _MB,),
    in_specs=[
        pl.BlockSpec((_MB, D_HID), lambda i: (i, 0)),
        pl.BlockSpec((_MB, 1), lambda i: (i, 0)),
        pl.BlockSpec((D_HID, D_OUT), lambda i: (0, 0)),
    ],
    out_specs=pl.BlockSpec((_MB, D_OUT), lambda i: (i, 0)),
    out_shape=jax.ShapeDtypeStruct((N, D_OUT), jnp.float32),
)


def _m3s_body(a_ref, d_ref, o_ref):
    o = a_ref[...] * d_ref[...]
    m = jnp.max(o, axis=1, keepdims=True)
    o_ref[...] = o - m - jnp.log(jnp.sum(jnp.exp(o - m), axis=1,
                                         keepdims=True))


_m3s = pl.pallas_call(
    _m3s_body,
    grid=(N // _MB,),
    in_specs=[
        pl.BlockSpec((_MB, D_OUT), lambda i: (i, 0)),
        pl.BlockSpec((_MB, 1), lambda i: (i, 0)),
    ],
    out_specs=pl.BlockSpec((_MB, D_OUT), lambda i: (i, 0)),
    out_shape=jax.ShapeDtypeStruct((N, D_OUT), jnp.float32),
)


def kernel(x, edge_index, W1, W2):
    src = edge_index[0].astype(jnp.int32)
    dst = edge_index[1].astype(jnp.int32)
    pad = jnp.full((E_PAD - E,), DUMMY, jnp.int32)
    src_g = jnp.concatenate([src, pad]).reshape(NC * NS, T_EDG, E_CH)
    dst_flat = jnp.concatenate([dst, pad])

    degj = jnp.zeros((N,), jnp.float32).at[dst].add(1.0)
    hist2 = jnp.stack([degj, jnp.zeros_like(degj)], axis=1)

    def _scatter(y, msgs):
        return y + jnp.zeros_like(y).at[dst_flat].add(
            msgs.reshape(E_PAD, -1)[:, :y.shape[1]], mode="drop")

    zpad = jnp.zeros((N_PAD - N, D_HID), jnp.float32)

    y1, dinv = _m1s(x, W1, hist2)
    msgs1 = _gat(jnp.concatenate([y1, zpad]), src_g)
    acc1 = _scatter(y1, msgs1)

    y2 = _m2s(acc1, dinv, W2)
    y2w = jnp.concatenate(
        [y2, jnp.zeros((N, D_HID - D_OUT), jnp.float32)], axis=1)
    msgs2 = _gat(jnp.concatenate([y2w, zpad]), src_g)
    acc2 = _scatter(y2, msgs2)

    return _m3s(acc2, dinv)
